# Initial kernel scaffold; baseline (speedup 1.0000x reference)
import jax
import jax.numpy as jnp
from jax.experimental import pallas as pl


def kernel(features, candidates, neighborhood_ids, obs_ix, means, Coo_inv, Coo_logdet, noise_log_prior):
    raise NotImplementedError("write your pallas kernel here")


# speedup vs baseline: 13.9665x; 13.9665x over previous
import math

import jax
import jax.numpy as jnp
from jax import lax
from jax.experimental import pallas as pl
from jax.experimental.pallas import tpu as pltpu
from jax.experimental.pallas import tpu_sc as plsc

S = 16384
RANK = 8
NC_OBS = 8
D = RANK * NC_OBS
N_UNITS = 512
N_CHANNELS = 384
N_NEIGHB = 32
N_CAND = 8
NRES = N_CAND + 1
WG = 64
WS = 80
ROWS = S * N_CAND
TROWS = N_NEIGHB * N_UNITS

NCORES = 2
NSUB = 16
NW = NCORES * NSUB
SPW = S // NW
CHUNK = 16
NCHUNK = SPW // CHUNK

_HI = jax.lax.Precision.HIGHEST
_LOG2PI = math.log(2.0 * math.pi)


def _a1_body(nid_ref, cand_ref, idx_ref):
    idx_ref[...] = nid_ref[...] * N_UNITS + cand_ref[...]


def _stage_a1(nid2, cand):
    return pl.pallas_call(
        _a1_body,
        grid=(4,),
        in_specs=[
            pl.BlockSpec((S // 4, 1), lambda i: (i, 0)),
            pl.BlockSpec((S // 4, N_CAND), lambda i: (i, 0)),
        ],
        out_specs=pl.BlockSpec((S // 4, N_CAND), lambda i: (i, 0)),
        out_shape=jax.ShapeDtypeStruct((S, N_CAND), jnp.int32),
    )(nid2, cand)


def _a2_body(means2_ref, obs_ref, t_ref):
    i0 = lax.broadcasted_iota(jnp.int32, (RANK * N_CHANNELS, D), 0)
    i1 = lax.broadcasted_iota(jnp.int32, (RANK * N_CHANNELS, D), 1)
    obs = obs_ref[0, 0, :]
    obs_col = jnp.tile(obs, RANK)
    sel = ((i0 // N_CHANNELS == i1 // NC_OBS)
           & (i0 % N_CHANNELS == obs_col[None, :])).astype(jnp.float32)
    t_ref[...] = jax.lax.dot_general(
        means2_ref[...], sel, (((1,), (0,)), ((), ())),
        precision=_HI, preferred_element_type=jnp.float32)


def _stage_a2(means2, obs3):
    return pl.pallas_call(
        _a2_body,
        grid=(N_NEIGHB,),
        in_specs=[
            pl.BlockSpec((N_UNITS, RANK * N_CHANNELS), lambda n: (0, 0)),
            pl.BlockSpec((1, 1, NC_OBS), lambda n: (n, 0, 0)),
        ],
        out_specs=pl.BlockSpec((N_UNITS, WG), lambda n: (n, 0)),
        out_shape=jax.ShapeDtypeStruct((TROWS, WG), jnp.float32),
    )(means2, obs3)


def _g_body(t_hbm, idx_hbm, out_hbm, idx_v, rows_v, sem):
    wid = lax.axis_index("s") * NCORES + lax.axis_index("c")
    base0 = wid * (SPW * N_CAND)
    for k in range(NCHUNK):
        base = base0 + k * (CHUNK * N_CAND)
        pltpu.sync_copy(idx_hbm.at[pl.ds(base, CHUNK * N_CAND)], idx_v)
        pltpu.async_copy(t_hbm.at[idx_v], rows_v, sem).wait()
        pltpu.sync_copy(rows_v, out_hbm.at[pl.ds(base, CHUNK * N_CAND)])


def _stage_g(table, idx_flat):
    mesh = plsc.VectorSubcoreMesh(core_axis_name="c", subcore_axis_name="s",
                                  num_cores=NCORES, num_subcores=NSUB)
    f = pl.kernel(
        _g_body,
        out_type=jax.ShapeDtypeStruct((ROWS, WG), jnp.float32),
        mesh=mesh,
        scratch_types=[
            pltpu.VMEM((CHUNK * N_CAND,), jnp.int32),
            pltpu.VMEM((CHUNK * N_CAND, WG), jnp.float32),
            pltpu.SemaphoreType.DMA,
        ],
        compiler_params=pltpu.CompilerParams(use_tc_tiling_on_sc=False),
    )
    return f(table, idx_flat)


def _b_body(rows_ref, x_ref, nid_ref, p_ref, ld_ref, prior_ref,
            w_ref, elbo_ref, nn_ref):
    i = pl.program_id(0)

    @pl.when(i == 0)
    def _():
        elbo_ref[...] = jnp.zeros_like(elbo_ref)
        nn_ref[...] = jnp.zeros_like(nn_ref)

    x = x_ref[...]
    nid = nid_ref[...]
    rows = rows_ref[...]
    bs = x.shape[0]

    rs = [x - rows[:, c * WG:(c + 1) * WG] for c in range(N_CAND)] + [x]
    r_all = jnp.concatenate(rs, axis=0)
    rb = r_all.astype(jnp.bfloat16)
    nid9 = jnp.concatenate([nid] * NRES, axis=0)

    acc = jnp.zeros((NRES * bs, D), jnp.float32)
    ldsel = jnp.zeros((bs, 1), jnp.float32)
    for n in range(N_NEIGHB):
        pb = p_ref[n].astype(jnp.bfloat16)
        pr = jax.lax.dot_general(rb, pb, (((1,), (0,)), ((), ())),
                                 preferred_element_type=jnp.float32)
        acc = acc + pr * (nid9 == n).astype(jnp.float32)
        ldsel = ldsel + ld_ref[0, n] * (nid == n).astype(jnp.float32)

    quad9 = jnp.sum(r_all * acc, axis=1, keepdims=True)
    const = -0.5 * (ldsel + D * _LOG2PI)
    lls = [-0.5 * quad9[c * bs:(c + 1) * bs] + const for c in range(NRES)]
    ll = jnp.concatenate(lls[:N_CAND], axis=1)
    ln = lls[N_CAND] + prior_ref[0, 0]

    mx = jnp.maximum(jnp.max(ll, axis=1, keepdims=True), ln)
    e = jnp.exp(ll - mx)
    en = jnp.exp(ln - mx)
    sm = jnp.sum(e, axis=1, keepdims=True) + en
    q = e / sm
    qn = en / sm
    lse = mx + jnp.log(sm)

    wparts = []
    padz = jnp.zeros((bs, WS - D - 1), jnp.float32)
    for c in range(N_CAND):
        qc = q[:, c:c + 1]
        wparts.append(jnp.concatenate([qc * x, qc, padz], axis=1))
    w_ref[...] = jnp.concatenate(wparts, axis=1)

    elbo_ref[...] += jnp.sum(lse) * (1.0 / S)
    nn_ref[...] += jnp.sum(qn)


def _stage_b(rows512, x, nid2, coo_inv, ld2, prior2):
    BS = 512
    grid = S // BS
    return pl.pallas_call(
        _b_body,
        grid=(grid,),
        in_specs=[
            pl.BlockSpec((BS, N_CAND * WG), lambda i: (i, 0)),
            pl.BlockSpec((BS, D), lambda i: (i, 0)),
            pl.BlockSpec((BS, 1), lambda i: (i, 0)),
            pl.BlockSpec((N_NEIGHB, D, D), lambda i: (0, 0, 0)),
            pl.BlockSpec((1, N_NEIGHB), lambda i: (0, 0)),
            pl.BlockSpec((1, 1), lambda i: (0, 0)),
        ],
        out_specs=[
            pl.BlockSpec((BS, N_CAND * WS), lambda i: (i, 0)),
            pl.BlockSpec((1, 1), lambda i: (0, 0)),
            pl.BlockSpec((1, 1), lambda i: (0, 0)),
        ],
        out_shape=[
            jax.ShapeDtypeStruct((S, N_CAND * WS), jnp.float32),
            jax.ShapeDtypeStruct((1, 1), jnp.float32),
            jax.ShapeDtypeStruct((1, 1), jnp.float32),
        ],
        compiler_params=pltpu.CompilerParams(
            vmem_limit_bytes=100 * 1024 * 1024),
    )(rows512, x, nid2, coo_inv, ld2, prior2)


def _s_body(w_hbm, idx_hbm, out_hbm, m_sh, idx_v, w_v, zbuf):
    c = lax.axis_index("c")
    sid = lax.axis_index("s")
    wid = sid * NCORES + c

    zero16 = jnp.zeros((16,), jnp.float32)

    def zrow(i, _):
        for j in range(WS // 16):
            zbuf[i, pl.ds(j * 16, 16)] = zero16
        return 0

    lax.fori_loop(0, CHUNK * N_CAND, zrow, 0)
    mrows = TROWS // NSUB
    for t in range(mrows // (CHUNK * N_CAND)):
        pltpu.sync_copy(zbuf, m_sh.at[pl.ds(sid * mrows + t * CHUNK * N_CAND,
                                            CHUNK * N_CAND)])
    plsc.subcore_barrier()

    base0 = wid * (SPW * N_CAND)
    for k in range(NCHUNK):
        base = base0 + k * (CHUNK * N_CAND)
        pltpu.sync_copy(idx_hbm.at[pl.ds(base, CHUNK * N_CAND)], idx_v)
        pltpu.sync_copy(w_hbm.at[pl.ds(base, CHUNK * N_CAND)], w_v)
        pltpu.sync_copy(w_v, m_sh.at[idx_v], add=True)
    plsc.subcore_barrier()

    for t in range(mrows // (CHUNK * N_CAND)):
        r0 = sid * mrows + t * CHUNK * N_CAND
        pltpu.sync_copy(m_sh.at[pl.ds(r0, CHUNK * N_CAND)],
                        out_hbm.at[c, pl.ds(r0, CHUNK * N_CAND)])


def _stage_s(wrows, idx_flat):
    mesh = plsc.VectorSubcoreMesh(core_axis_name="c", subcore_axis_name="s",
                                  num_cores=NCORES, num_subcores=NSUB)
    f = pl.kernel(
        _s_body,
        out_type=jax.ShapeDtypeStruct((NCORES, TROWS, WS), jnp.float32),
        mesh=mesh,
        scratch_types=[
            pltpu.VMEM_SHARED((TROWS, WS), jnp.float32),
            pltpu.VMEM((CHUNK * N_CAND,), jnp.int32),
            pltpu.VMEM((CHUNK * N_CAND, WS), jnp.float32),
            pltpu.VMEM((CHUNK * N_CAND, WS), jnp.float32),
        ],
        compiler_params=pltpu.CompilerParams(use_tc_tiling_on_sc=False),
    )
    return f(wrows, idx_flat)


def _c_body(m_ref, obs_ref, m2_ref, n_ref):
    n = pl.program_id(0)

    @pl.when(n == 0)
    def _():
        m2_ref[...] = jnp.zeros_like(m2_ref)
        n_ref[...] = jnp.zeros_like(n_ref)

    msum = m_ref[0] + m_ref[1]
    n_ref[...] += msum[:, D:D + 1]

    i0 = lax.broadcasted_iota(jnp.int32, (D, RANK * N_CHANNELS), 0)
    i1 = lax.broadcasted_iota(jnp.int32, (D, RANK * N_CHANNELS), 1)
    obs = obs_ref[0, 0, :]
    obs_col = jnp.tile(obs, RANK)
    sel = ((i0 // NC_OBS == i1 // N_CHANNELS)
           & (obs_col[:, None] == i1 % N_CHANNELS)).astype(jnp.float32)
    m2_ref[...] += jax.lax.dot_general(
        msum[:, :D], sel, (((1,), (0,)), ((), ())),
        precision=_HI, preferred_element_type=jnp.float32)

    @pl.when(n == N_NEIGHB - 1)
    def _():
        nv = n_ref[...]
        den = jnp.where(nv == 0.0, 1.0, nv)
        m2_ref[...] = m2_ref[...] / den


def _stage_c(mparts, obs3):
    return pl.pallas_call(
        _c_body,
        grid=(N_NEIGHB,),
        in_specs=[
            pl.BlockSpec((NCORES, N_UNITS, WS), lambda n: (0, n, 0)),
            pl.BlockSpec((1, 1, NC_OBS), lambda n: (n, 0, 0)),
        ],
        out_specs=[
            pl.BlockSpec((N_UNITS, RANK * N_CHANNELS), lambda n: (0, 0)),
            pl.BlockSpec((N_UNITS, 1), lambda n: (0, 0)),
        ],
        out_shape=[
            jax.ShapeDtypeStruct((N_UNITS, RANK * N_CHANNELS), jnp.float32),
            jax.ShapeDtypeStruct((N_UNITS, 1), jnp.float32),
        ],
    )(mparts, obs3)


def kernel(features, candidates, neighborhood_ids, obs_ix, means, Coo_inv,
           Coo_logdet, noise_log_prior):
    x = features.reshape(S, D)
    nid2 = neighborhood_ids.reshape(S, 1).astype(jnp.int32)
    cand = candidates.astype(jnp.int32)
    ld2 = Coo_logdet.reshape(1, N_NEIGHB)
    means2 = means.reshape(N_UNITS, RANK * N_CHANNELS)
    obs3 = obs_ix.reshape(N_NEIGHB, 1, NC_OBS).astype(jnp.int32)
    prior2 = noise_log_prior.reshape(1, 1)

    idx = _stage_a1(nid2, cand)
    table = _stage_a2(means2, obs3)
    idx_flat = idx.reshape(ROWS)
    rows = _stage_g(table, idx_flat)
    w640, elbo, nn = _stage_b(rows.reshape(S, N_CAND * WG), x, nid2, Coo_inv,
                              ld2, prior2)
    mparts = _stage_s(w640.reshape(ROWS, WS), idx_flat)
    m2, nvec = _stage_c(mparts, obs3)

    obs_elbo = elbo.reshape(())
    noise_n = nn.reshape(())
    return obs_elbo, nvec.reshape(N_UNITS), m2.reshape(N_UNITS, RANK,
                                                       N_CHANNELS), noise_n


# speedup vs baseline: 19.7688x; 1.4154x over previous
import math

import jax
import jax.numpy as jnp
from jax import lax
from jax.experimental import pallas as pl
from jax.experimental.pallas import tpu as pltpu
from jax.experimental.pallas import tpu_sc as plsc

S = 16384
RANK = 8
NC_OBS = 8
D = RANK * NC_OBS
N_UNITS = 512
N_CHANNELS = 384
N_NEIGHB = 32
N_CAND = 8
NRES = N_CAND + 1
WG = 64
WS = 80
ROWS = S * N_CAND
TROWS = N_NEIGHB * N_UNITS

NCORES = 2
NSUB = 16
NW = NCORES * NSUB
SPW = S // NW
CHUNK = 16
NCHUNK = SPW // CHUNK

_HI = jax.lax.Precision.HIGHEST
_LOG2PI = math.log(2.0 * math.pi)


def _a1_body(nid_ref, cand_ref, idx_ref):
    idx_ref[...] = nid_ref[...] * N_UNITS + cand_ref[...]


def _stage_a1(nid2, cand):
    return pl.pallas_call(
        _a1_body,
        grid=(4,),
        in_specs=[
            pl.BlockSpec((S // 4, 1), lambda i: (i, 0)),
            pl.BlockSpec((S // 4, N_CAND), lambda i: (i, 0)),
        ],
        out_specs=pl.BlockSpec((S // 4, N_CAND), lambda i: (i, 0)),
        out_shape=jax.ShapeDtypeStruct((S, N_CAND), jnp.int32),
    )(nid2, cand)


def _a2_body(means2_ref, obs_ref, t_ref):
    i0 = lax.broadcasted_iota(jnp.int32, (RANK * N_CHANNELS, D), 0)
    i1 = lax.broadcasted_iota(jnp.int32, (RANK * N_CHANNELS, D), 1)
    obs = obs_ref[0, 0, :]
    obs_col = jnp.tile(obs, RANK)
    sel = ((i0 // N_CHANNELS == i1 // NC_OBS)
           & (i0 % N_CHANNELS == obs_col[None, :])).astype(jnp.bfloat16)
    mns = means2_ref[...]
    mhi = mns.astype(jnp.bfloat16)
    mlo = (mns - mhi.astype(jnp.float32)).astype(jnp.bfloat16)
    dn = (((1,), (0,)), ((), ()))
    t_ref[...] = (
        jax.lax.dot_general(mhi, sel, dn, preferred_element_type=jnp.float32)
        + jax.lax.dot_general(mlo, sel, dn,
                              preferred_element_type=jnp.float32))


def _stage_a2(means2, obs3):
    return pl.pallas_call(
        _a2_body,
        grid=(N_NEIGHB,),
        in_specs=[
            pl.BlockSpec((N_UNITS, RANK * N_CHANNELS), lambda n: (0, 0)),
            pl.BlockSpec((1, 1, NC_OBS), lambda n: (n, 0, 0)),
        ],
        out_specs=pl.BlockSpec((N_UNITS, WG), lambda n: (n, 0)),
        out_shape=jax.ShapeDtypeStruct((TROWS, WG), jnp.float32),
    )(means2, obs3)


def _g_body(t_hbm, idx_hbm, out_hbm, idx_v, rows_v, sem):
    wid = lax.axis_index("s") * NCORES + lax.axis_index("c")
    base0 = wid * (SPW * N_CAND)
    for k in range(NCHUNK):
        base = base0 + k * (CHUNK * N_CAND)
        pltpu.sync_copy(idx_hbm.at[pl.ds(base, CHUNK * N_CAND)], idx_v)
        pltpu.async_copy(t_hbm.at[idx_v], rows_v, sem).wait()
        pltpu.sync_copy(rows_v, out_hbm.at[pl.ds(base, CHUNK * N_CAND)])


def _stage_g(table, idx_flat):
    mesh = plsc.VectorSubcoreMesh(core_axis_name="c", subcore_axis_name="s",
                                  num_cores=NCORES, num_subcores=NSUB)
    f = pl.kernel(
        _g_body,
        out_type=jax.ShapeDtypeStruct((ROWS, WG), jnp.float32),
        mesh=mesh,
        scratch_types=[
            pltpu.VMEM((CHUNK * N_CAND,), jnp.int32),
            pltpu.VMEM((CHUNK * N_CAND, WG), jnp.float32),
            pltpu.SemaphoreType.DMA,
        ],
        compiler_params=pltpu.CompilerParams(use_tc_tiling_on_sc=False),
    )
    return f(table, idx_flat)


def _b_body(rows_ref, x_ref, nid_ref, pstack_ref, ld_ref, prior_ref,
            w_ref, elbo_ref, nn_ref):
    i = pl.program_id(0)

    @pl.when(i == 0)
    def _():
        elbo_ref[...] = jnp.zeros_like(elbo_ref)
        nn_ref[...] = jnp.zeros_like(nn_ref)

    x = x_ref[...]
    nid = nid_ref[...]
    rows = rows_ref[...]
    bs = x.shape[0]

    rs = [x - rows[:, c * WG:(c + 1) * WG] for c in range(N_CAND)] + [x]
    r_all = jnp.concatenate(rs, axis=0)
    rb = r_all.astype(jnp.bfloat16)
    nid9 = jnp.concatenate([nid] * NRES, axis=0)

    zero_b = jnp.zeros((), jnp.bfloat16)
    chunks = [jnp.where(nid9 == n, rb, zero_b) for n in range(N_NEIGHB)]
    rwide = jnp.concatenate(chunks, axis=1)
    acc = jax.lax.dot_general(
        rwide, pstack_ref[...].astype(jnp.bfloat16), (((1,), (0,)), ((), ())),
        preferred_element_type=jnp.float32)

    ldsel = jnp.zeros((bs, 1), jnp.float32)
    for n in range(N_NEIGHB):
        ldsel = ldsel + ld_ref[0, n] * (nid == n).astype(jnp.float32)

    quad9 = jnp.sum(r_all * acc, axis=1, keepdims=True)
    const = -0.5 * (ldsel + D * _LOG2PI)
    lls = [-0.5 * quad9[c * bs:(c + 1) * bs] + const for c in range(NRES)]
    ll = jnp.concatenate(lls[:N_CAND], axis=1)
    ln = lls[N_CAND] + prior_ref[0, 0]

    mx = jnp.maximum(jnp.max(ll, axis=1, keepdims=True), ln)
    e = jnp.exp(ll - mx)
    en = jnp.exp(ln - mx)
    sm = jnp.sum(e, axis=1, keepdims=True) + en
    q = e / sm
    qn = en / sm
    lse = mx + jnp.log(sm)

    wparts = []
    padz = jnp.zeros((bs, WS - D - 1), jnp.float32)
    for c in range(N_CAND):
        qc = q[:, c:c + 1]
        wparts.append(jnp.concatenate([qc * x, qc, padz], axis=1))
    w_ref[...] = jnp.concatenate(wparts, axis=1)

    elbo_ref[...] += jnp.sum(lse) * (1.0 / S)
    nn_ref[...] += jnp.sum(qn)


def _stage_b(rows512, x, nid2, pstack, ld2, prior2):
    BS = 512
    grid = S // BS
    return pl.pallas_call(
        _b_body,
        grid=(grid,),
        in_specs=[
            pl.BlockSpec((BS, N_CAND * WG), lambda i: (i, 0)),
            pl.BlockSpec((BS, D), lambda i: (i, 0)),
            pl.BlockSpec((BS, 1), lambda i: (i, 0)),
            pl.BlockSpec((N_NEIGHB * D, D), lambda i: (0, 0)),
            pl.BlockSpec((1, N_NEIGHB), lambda i: (0, 0)),
            pl.BlockSpec((1, 1), lambda i: (0, 0)),
        ],
        out_specs=[
            pl.BlockSpec((BS, N_CAND * WS), lambda i: (i, 0)),
            pl.BlockSpec((1, 1), lambda i: (0, 0)),
            pl.BlockSpec((1, 1), lambda i: (0, 0)),
        ],
        out_shape=[
            jax.ShapeDtypeStruct((S, N_CAND * WS), jnp.float32),
            jax.ShapeDtypeStruct((1, 1), jnp.float32),
            jax.ShapeDtypeStruct((1, 1), jnp.float32),
        ],
        compiler_params=pltpu.CompilerParams(
            vmem_limit_bytes=100 * 1024 * 1024),
    )(rows512, x, nid2, pstack, ld2, prior2)


def _s_body(w_hbm, idx_hbm, out_hbm, m_sh, idx_v, w_v, zbuf):
    c = lax.axis_index("c")
    sid = lax.axis_index("s")
    wid = sid * NCORES + c

    zero16 = jnp.zeros((16,), jnp.float32)

    def zrow(i, _):
        for j in range(WS // 16):
            zbuf[i, pl.ds(j * 16, 16)] = zero16
        return 0

    lax.fori_loop(0, CHUNK * N_CAND, zrow, 0)
    mrows = TROWS // NSUB
    for t in range(mrows // (CHUNK * N_CAND)):
        pltpu.sync_copy(zbuf, m_sh.at[pl.ds(sid * mrows + t * CHUNK * N_CAND,
                                            CHUNK * N_CAND)])
    plsc.subcore_barrier()

    base0 = wid * (SPW * N_CAND)
    for k in range(NCHUNK):
        base = base0 + k * (CHUNK * N_CAND)
        pltpu.sync_copy(idx_hbm.at[pl.ds(base, CHUNK * N_CAND)], idx_v)
        pltpu.sync_copy(w_hbm.at[pl.ds(base, CHUNK * N_CAND)], w_v)
        pltpu.sync_copy(w_v, m_sh.at[idx_v], add=True)
    plsc.subcore_barrier()

    for t in range(mrows // (CHUNK * N_CAND)):
        r0 = sid * mrows + t * CHUNK * N_CAND
        pltpu.sync_copy(m_sh.at[pl.ds(r0, CHUNK * N_CAND)],
                        out_hbm.at[c, pl.ds(r0, CHUNK * N_CAND)])


def _stage_s(wrows, idx_flat):
    mesh = plsc.VectorSubcoreMesh(core_axis_name="c", subcore_axis_name="s",
                                  num_cores=NCORES, num_subcores=NSUB)
    f = pl.kernel(
        _s_body,
        out_type=jax.ShapeDtypeStruct((NCORES, TROWS, WS), jnp.float32),
        mesh=mesh,
        scratch_types=[
            pltpu.VMEM_SHARED((TROWS, WS), jnp.float32),
            pltpu.VMEM((CHUNK * N_CAND,), jnp.int32),
            pltpu.VMEM((CHUNK * N_CAND, WS), jnp.float32),
            pltpu.VMEM((CHUNK * N_CAND, WS), jnp.float32),
        ],
        compiler_params=pltpu.CompilerParams(use_tc_tiling_on_sc=False),
    )
    return f(wrows, idx_flat)


def _c_body(m_ref, obs_ref, m2_ref, n_ref):
    n = pl.program_id(0)

    @pl.when(n == 0)
    def _():
        m2_ref[...] = jnp.zeros_like(m2_ref)
        n_ref[...] = jnp.zeros_like(n_ref)

    msum = m_ref[0] + m_ref[1]
    n_ref[...] += msum[:, D:D + 1]

    i0 = lax.broadcasted_iota(jnp.int32, (D, RANK * N_CHANNELS), 0)
    i1 = lax.broadcasted_iota(jnp.int32, (D, RANK * N_CHANNELS), 1)
    obs = obs_ref[0, 0, :]
    obs_col = jnp.tile(obs, RANK)
    sel = ((i0 // NC_OBS == i1 // N_CHANNELS)
           & (obs_col[:, None] == i1 % N_CHANNELS)).astype(jnp.bfloat16)
    mhi = msum[:, :D].astype(jnp.bfloat16)
    mlo = (msum[:, :D] - mhi.astype(jnp.float32)).astype(jnp.bfloat16)
    dn = (((1,), (0,)), ((), ()))
    m2_ref[...] += (
        jax.lax.dot_general(mhi, sel, dn, preferred_element_type=jnp.float32)
        + jax.lax.dot_general(mlo, sel, dn,
                              preferred_element_type=jnp.float32))

    @pl.when(n == N_NEIGHB - 1)
    def _():
        nv = n_ref[...]
        den = jnp.where(nv == 0.0, 1.0, nv)
        m2_ref[...] = m2_ref[...] / den


def _stage_c(mparts, obs3):
    return pl.pallas_call(
        _c_body,
        grid=(N_NEIGHB,),
        in_specs=[
            pl.BlockSpec((NCORES, N_UNITS, WS), lambda n: (0, n, 0)),
            pl.BlockSpec((1, 1, NC_OBS), lambda n: (n, 0, 0)),
        ],
        out_specs=[
            pl.BlockSpec((N_UNITS, RANK * N_CHANNELS), lambda n: (0, 0)),
            pl.BlockSpec((N_UNITS, 1), lambda n: (0, 0)),
        ],
        out_shape=[
            jax.ShapeDtypeStruct((N_UNITS, RANK * N_CHANNELS), jnp.float32),
            jax.ShapeDtypeStruct((N_UNITS, 1), jnp.float32),
        ],
    )(mparts, obs3)


def kernel(features, candidates, neighborhood_ids, obs_ix, means, Coo_inv,
           Coo_logdet, noise_log_prior):
    x = features.reshape(S, D)
    nid2 = neighborhood_ids.reshape(S, 1).astype(jnp.int32)
    cand = candidates.astype(jnp.int32)
    ld2 = Coo_logdet.reshape(1, N_NEIGHB)
    means2 = means.reshape(N_UNITS, RANK * N_CHANNELS)
    obs3 = obs_ix.reshape(N_NEIGHB, 1, NC_OBS).astype(jnp.int32)
    prior2 = noise_log_prior.reshape(1, 1)

    idx = _stage_a1(nid2, cand)
    table = _stage_a2(means2, obs3)
    idx_flat = idx.reshape(ROWS)
    rows = _stage_g(table, idx_flat)
    pstack = Coo_inv.reshape(N_NEIGHB * D, D)
    w640, elbo, nn = _stage_b(rows.reshape(S, N_CAND * WG), x, nid2, pstack,
                              ld2, prior2)
    mparts = _stage_s(w640.reshape(ROWS, WS), idx_flat)
    m2, nvec = _stage_c(mparts, obs3)

    obs_elbo = elbo.reshape(())
    noise_n = nn.reshape(())
    return obs_elbo, nvec.reshape(N_UNITS), m2.reshape(N_UNITS, RANK,
                                                       N_CHANNELS), noise_n


# speedup vs baseline: 19.7903x; 1.0011x over previous
import math

import jax
import jax.numpy as jnp
from jax import lax
from jax.experimental import pallas as pl
from jax.experimental.pallas import tpu as pltpu
from jax.experimental.pallas import tpu_sc as plsc

S = 16384
RANK = 8
NC_OBS = 8
D = RANK * NC_OBS
N_UNITS = 512
N_CHANNELS = 384
N_NEIGHB = 32
N_CAND = 8
NRES = N_CAND + 1
WG = 64
WS = 80
ROWS = S * N_CAND
TROWS = N_NEIGHB * N_UNITS

NCORES = 2
NSUB = 16
NW = NCORES * NSUB
SPW = S // NW
CHUNK = 16
NCHUNK = SPW // CHUNK

_LOG2PI = math.log(2.0 * math.pi)


def _a1_body(nid_ref, cand_ref, idx_ref):
    idx_ref[...] = nid_ref[...] * N_UNITS + cand_ref[...]


def _stage_a1(nid2, cand):
    return pl.pallas_call(
        _a1_body,
        grid=(4,),
        in_specs=[
            pl.BlockSpec((S // 4, 1), lambda i: (i, 0)),
            pl.BlockSpec((S // 4, N_CAND), lambda i: (i, 0)),
        ],
        out_specs=pl.BlockSpec((S // 4, N_CAND), lambda i: (i, 0)),
        out_shape=jax.ShapeDtypeStruct((S, N_CAND), jnp.int32),
    )(nid2, cand)


def _a2_body(means2_ref, obs_ref, t_ref):
    i0 = lax.broadcasted_iota(jnp.int32, (RANK * N_CHANNELS, D), 0)
    i1 = lax.broadcasted_iota(jnp.int32, (RANK * N_CHANNELS, D), 1)
    obs = obs_ref[0, 0, :]
    obs_col = jnp.tile(obs, RANK)
    sel = ((i0 // N_CHANNELS == i1 // NC_OBS)
           & (i0 % N_CHANNELS == obs_col[None, :])).astype(jnp.bfloat16)
    mns = means2_ref[...]
    mhi = mns.astype(jnp.bfloat16)
    mlo = (mns - mhi.astype(jnp.float32)).astype(jnp.bfloat16)
    dn = (((1,), (0,)), ((), ()))
    t_ref[...] = (
        jax.lax.dot_general(mhi, sel, dn, preferred_element_type=jnp.float32)
        + jax.lax.dot_general(mlo, sel, dn,
                              preferred_element_type=jnp.float32))


def _stage_a2(means2, obs3):
    return pl.pallas_call(
        _a2_body,
        grid=(N_NEIGHB,),
        in_specs=[
            pl.BlockSpec((N_UNITS, RANK * N_CHANNELS), lambda n: (0, 0)),
            pl.BlockSpec((1, 1, NC_OBS), lambda n: (n, 0, 0)),
        ],
        out_specs=pl.BlockSpec((N_UNITS, WG), lambda n: (n, 0)),
        out_shape=jax.ShapeDtypeStruct((TROWS, WG), jnp.float32),
    )(means2, obs3)


def _g_body(t_hbm, idx_hbm, out_hbm, idx_v, rows_v, sem):
    wid = lax.axis_index("s") * NCORES + lax.axis_index("c")
    base0 = wid * (SPW * N_CAND)
    for k in range(NCHUNK):
        base = base0 + k * (CHUNK * N_CAND)
        pltpu.sync_copy(idx_hbm.at[pl.ds(base, CHUNK * N_CAND)], idx_v)
        pltpu.async_copy(t_hbm.at[idx_v], rows_v, sem).wait()
        pltpu.sync_copy(rows_v, out_hbm.at[pl.ds(base, CHUNK * N_CAND)])


def _stage_g(table, idx_flat):
    mesh = plsc.VectorSubcoreMesh(core_axis_name="c", subcore_axis_name="s",
                                  num_cores=NCORES, num_subcores=NSUB)
    f = pl.kernel(
        _g_body,
        out_type=jax.ShapeDtypeStruct((ROWS, WG), jnp.float32),
        mesh=mesh,
        scratch_types=[
            pltpu.VMEM((CHUNK * N_CAND,), jnp.int32),
            pltpu.VMEM((CHUNK * N_CAND, WG), jnp.float32),
            pltpu.SemaphoreType.DMA,
        ],
        compiler_params=pltpu.CompilerParams(use_tc_tiling_on_sc=False),
    )
    return f(table, idx_flat)


def _b_body(rows_ref, x_ref, nid_ref, pstack_ref, ld_ref, prior_ref,
            w_ref, elbo_ref, nn_ref):
    i = pl.program_id(0)

    @pl.when(i == 0)
    def _():
        elbo_ref[...] = jnp.zeros_like(elbo_ref)
        nn_ref[...] = jnp.zeros_like(nn_ref)

    x = x_ref[...]
    nid = nid_ref[...]
    rows = rows_ref[...]
    bs = x.shape[0]

    rs = [x - rows[:, c * WG:(c + 1) * WG] for c in range(N_CAND)] + [x]
    r_all = jnp.concatenate(rs, axis=0)
    rb = r_all.astype(jnp.bfloat16)
    nid9 = jnp.concatenate([nid] * NRES, axis=0)

    zero_b = jnp.zeros((), jnp.bfloat16)
    chunks = [jnp.where(nid9 == n, rb, zero_b) for n in range(N_NEIGHB)]
    rwide = jnp.concatenate(chunks, axis=1)
    acc = jax.lax.dot_general(
        rwide, pstack_ref[...].astype(jnp.bfloat16), (((1,), (0,)), ((), ())),
        preferred_element_type=jnp.float32)

    ldsel = jnp.zeros((bs, 1), jnp.float32)
    for n in range(N_NEIGHB):
        ldsel = ldsel + ld_ref[0, n] * (nid == n).astype(jnp.float32)

    quad9 = jnp.sum(r_all * acc, axis=1, keepdims=True)
    const = -0.5 * (ldsel + D * _LOG2PI)
    lls = [-0.5 * quad9[c * bs:(c + 1) * bs] + const for c in range(NRES)]
    ll = jnp.concatenate(lls[:N_CAND], axis=1)
    ln = lls[N_CAND] + prior_ref[0, 0]

    mx = jnp.maximum(jnp.max(ll, axis=1, keepdims=True), ln)
    e = jnp.exp(ll - mx)
    en = jnp.exp(ln - mx)
    sm = jnp.sum(e, axis=1, keepdims=True) + en
    q = e / sm
    qn = en / sm
    lse = mx + jnp.log(sm)

    wparts = []
    padz = jnp.zeros((bs, WS - D - 1), jnp.float32)
    for c in range(N_CAND):
        qc = q[:, c:c + 1]
        wparts.append(jnp.concatenate([qc * x, qc, padz], axis=1))
    w_ref[...] = jnp.concatenate(wparts, axis=1)

    elbo_ref[...] += jnp.sum(lse) * (1.0 / S)
    nn_ref[...] += jnp.sum(qn)


def _stage_b(rows512, x, nid2, pstack, ld2, prior2):
    BS = 512
    grid = S // BS
    return pl.pallas_call(
        _b_body,
        grid=(grid,),
        in_specs=[
            pl.BlockSpec((BS, N_CAND * WG), lambda i: (i, 0)),
            pl.BlockSpec((BS, D), lambda i: (i, 0)),
            pl.BlockSpec((BS, 1), lambda i: (i, 0)),
            pl.BlockSpec((N_NEIGHB * D, D), lambda i: (0, 0)),
            pl.BlockSpec((1, N_NEIGHB), lambda i: (0, 0)),
            pl.BlockSpec((1, 1), lambda i: (0, 0)),
        ],
        out_specs=[
            pl.BlockSpec((BS, N_CAND * WS), lambda i: (i, 0)),
            pl.BlockSpec((1, 1), lambda i: (0, 0)),
            pl.BlockSpec((1, 1), lambda i: (0, 0)),
        ],
        out_shape=[
            jax.ShapeDtypeStruct((S, N_CAND * WS), jnp.float32),
            jax.ShapeDtypeStruct((1, 1), jnp.float32),
            jax.ShapeDtypeStruct((1, 1), jnp.float32),
        ],
        compiler_params=pltpu.CompilerParams(
            vmem_limit_bytes=100 * 1024 * 1024),
    )(rows512, x, nid2, pstack, ld2, prior2)


def _s_body(w_hbm, idx_hbm, out_hbm, m_sh, idx_v, w_v, zbuf):
    c = lax.axis_index("c")
    sid = lax.axis_index("s")
    wid = sid * NCORES + c

    zero16 = jnp.zeros((16,), jnp.float32)

    def zrow(i, _):
        for j in range(WS // 16):
            zbuf[i, pl.ds(j * 16, 16)] = zero16
        return 0

    lax.fori_loop(0, CHUNK * N_CAND, zrow, 0)
    mrows = TROWS // NSUB
    for t in range(mrows // (CHUNK * N_CAND)):
        pltpu.sync_copy(zbuf, m_sh.at[pl.ds(sid * mrows + t * CHUNK * N_CAND,
                                            CHUNK * N_CAND)])
    plsc.subcore_barrier()

    base0 = wid * (SPW * N_CAND)
    for k in range(NCHUNK):
        base = base0 + k * (CHUNK * N_CAND)
        pltpu.sync_copy(idx_hbm.at[pl.ds(base, CHUNK * N_CAND)], idx_v)
        pltpu.sync_copy(w_hbm.at[pl.ds(base, CHUNK * N_CAND)], w_v)
        pltpu.sync_copy(w_v, m_sh.at[idx_v], add=True)
    plsc.subcore_barrier()

    for t in range(mrows // (CHUNK * N_CAND)):
        r0 = sid * mrows + t * CHUNK * N_CAND
        pltpu.sync_copy(m_sh.at[pl.ds(r0, CHUNK * N_CAND)],
                        out_hbm.at[c, pl.ds(r0, CHUNK * N_CAND)])


def _stage_s(wrows, idx_flat):
    mesh = plsc.VectorSubcoreMesh(core_axis_name="c", subcore_axis_name="s",
                                  num_cores=NCORES, num_subcores=NSUB)
    f = pl.kernel(
        _s_body,
        out_type=jax.ShapeDtypeStruct((NCORES, TROWS, WS), jnp.float32),
        mesh=mesh,
        scratch_types=[
            pltpu.VMEM_SHARED((TROWS, WS), jnp.float32),
            pltpu.VMEM((CHUNK * N_CAND,), jnp.int32),
            pltpu.VMEM((CHUNK * N_CAND, WS), jnp.float32),
            pltpu.VMEM((CHUNK * N_CAND, WS), jnp.float32),
        ],
        compiler_params=pltpu.CompilerParams(use_tc_tiling_on_sc=False),
    )
    return f(wrows, idx_flat)


def _c_body(m_ref, obs_ref, m2_ref, n_ref):
    n = pl.program_id(0)

    @pl.when(n == 0)
    def _():
        m2_ref[...] = jnp.zeros_like(m2_ref)
        n_ref[...] = jnp.zeros_like(n_ref)

    msum = m_ref[0] + m_ref[1]
    n_ref[...] += msum[:, D:D + 1]

    i0 = lax.broadcasted_iota(jnp.int32, (D, RANK * N_CHANNELS), 0)
    i1 = lax.broadcasted_iota(jnp.int32, (D, RANK * N_CHANNELS), 1)
    obs = obs_ref[0, 0, :]
    obs_col = jnp.tile(obs, RANK)
    sel = ((i0 // NC_OBS == i1 // N_CHANNELS)
           & (obs_col[:, None] == i1 % N_CHANNELS)).astype(jnp.bfloat16)
    mhi = msum[:, :D].astype(jnp.bfloat16)
    mlo = (msum[:, :D] - mhi.astype(jnp.float32)).astype(jnp.bfloat16)
    dn = (((1,), (0,)), ((), ()))
    m2_ref[...] += (
        jax.lax.dot_general(mhi, sel, dn, preferred_element_type=jnp.float32)
        + jax.lax.dot_general(mlo, sel, dn,
                              preferred_element_type=jnp.float32))

    @pl.when(n == N_NEIGHB - 1)
    def _():
        nv = n_ref[...]
        den = jnp.where(nv == 0.0, 1.0, nv)
        m2_ref[...] = m2_ref[...] / den


def _stage_c(mparts, obs3):
    return pl.pallas_call(
        _c_body,
        grid=(N_NEIGHB,),
        in_specs=[
            pl.BlockSpec((NCORES, N_UNITS, WS), lambda n: (0, n, 0)),
            pl.BlockSpec((1, 1, NC_OBS), lambda n: (n, 0, 0)),
        ],
        out_specs=[
            pl.BlockSpec((N_UNITS, RANK * N_CHANNELS), lambda n: (0, 0)),
            pl.BlockSpec((N_UNITS, 1), lambda n: (0, 0)),
        ],
        out_shape=[
            jax.ShapeDtypeStruct((N_UNITS, RANK * N_CHANNELS), jnp.float32),
            jax.ShapeDtypeStruct((N_UNITS, 1), jnp.float32),
        ],
    )(mparts, obs3)


def kernel(features, candidates, neighborhood_ids, obs_ix, means, Coo_inv,
           Coo_logdet, noise_log_prior):
    x = features.reshape(S, D)
    nid2 = neighborhood_ids.reshape(S, 1).astype(jnp.int32)
    cand = candidates.astype(jnp.int32)
    ld2 = Coo_logdet.reshape(1, N_NEIGHB)
    means2 = means.reshape(N_UNITS, RANK * N_CHANNELS)
    obs3 = obs_ix.reshape(N_NEIGHB, 1, NC_OBS).astype(jnp.int32)
    prior2 = noise_log_prior.reshape(1, 1)

    idx = _stage_a1(nid2, cand)
    table = _stage_a2(means2, obs3)
    idx_flat = idx.reshape(ROWS)
    rows = _stage_g(table, idx_flat)
    pstack = Coo_inv.reshape(N_NEIGHB * D, D)
    w640, elbo, nn = _stage_b(rows.reshape(S, N_CAND * WG), x, nid2, pstack,
                              ld2, prior2)
    mparts = _stage_s(w640.reshape(ROWS, WS), idx_flat)
    m2, nvec = _stage_c(mparts, obs3)

    obs_elbo = elbo.reshape(())
    noise_n = nn.reshape(())
    return obs_elbo, nvec.reshape(N_UNITS), m2.reshape(N_UNITS, RANK,
                                                       N_CHANNELS), noise_n
